# baseline (speedup 1.0000x reference)
import functools

import jax
import jax.numpy as jnp
from jax import lax
from jax.experimental import pallas as pl
from jax.experimental.pallas import tpu as pltpu
from jax.experimental.pallas import tpu_sc as plsc

_IDX_CHUNK = 128


def _make_sc_gather(num_cores: int, b_per_w: int, dim: int, batch: int, dtype):
    k_chunks = b_per_w // _IDX_CHUNK
    mesh = plsc.VectorSubcoreMesh(core_axis_name="c", subcore_axis_name="s")

    @functools.partial(
        pl.kernel,
        mesh=mesh,
        compiler_params=pltpu.CompilerParams(use_tc_tiling_on_sc=False),
        out_type=jax.ShapeDtypeStruct((batch, dim), dtype),
        scratch_types=[
            pltpu.VMEM((k_chunks, _IDX_CHUNK), jnp.int32),
            pltpu.VMEM((b_per_w, dim), dtype),
            pltpu.SemaphoreType.DMA,
        ],
    )
    def gather_kernel(table_hbm, idx_hbm, out_hbm, idx_v, rows_v, sem):
        wid = lax.axis_index("s") * num_cores + lax.axis_index("c")
        base = wid * b_per_w
        pltpu.sync_copy(idx_hbm.at[wid], idx_v)
        copies = []
        for j in range(k_chunks):
            copies.append(
                pltpu.async_copy(
                    table_hbm.at[idx_v.at[j]],
                    rows_v.at[pl.ds(j * _IDX_CHUNK, _IDX_CHUNK)],
                    sem,
                ))
        for c in copies:
            c.wait()
        pltpu.sync_copy(rows_v, out_hbm.at[pl.ds(base, b_per_w)])

    return gather_kernel


def kernel(indices, table):
    (batch,) = indices.shape
    _, dim = table.shape
    info = plsc.get_sparse_core_info()
    num_workers = info.num_cores * info.num_subcores
    b_per_w = batch // num_workers
    idx3 = indices.astype(jnp.int32).reshape(
        num_workers, b_per_w // _IDX_CHUNK, _IDX_CHUNK)
    fn = _make_sc_gather(info.num_cores, b_per_w, dim, batch, table.dtype)
    return fn(table, idx3)


# speedup vs baseline: 2.4179x; 2.4179x over previous
import functools

import jax
import jax.numpy as jnp
from jax import lax
from jax.experimental import pallas as pl
from jax.experimental.pallas import tpu as pltpu
from jax.experimental.pallas import tpu_sc as plsc

_LANES = 16


def _make_sc_gather(num_cores: int, num_workers: int, b_per_w: int):
    mesh = plsc.VectorSubcoreMesh(core_axis_name="c", subcore_axis_name="s")
    batch = num_workers * b_per_w

    @functools.partial(
        pl.kernel,
        mesh=mesh,
        out_type=jax.ShapeDtypeStruct((batch, 128), jnp.float32),
        scratch_types=[
            pltpu.VMEM((8, 128), jnp.int32),
            pltpu.VMEM((b_per_w, 128), jnp.float32),
            pltpu.SemaphoreType.DMA,
        ],
    )
    def gather_kernel(tiles_hbm, idx_hbm, out_hbm, idx_v, outb, sem):
        wid = lax.axis_index("s") * num_cores + lax.axis_index("c")
        base = wid * b_per_w
        pltpu.sync_copy(idx_hbm.at[wid], idx_v)
        copies = []
        for g in range(b_per_w // _LANES):
            ivec = idx_v[g // 8, pl.ds(_LANES * (g % 8), _LANES)]
            tvec = ivec >> 3
            svec = ivec & 7
            for l in range(_LANES):
                j = _LANES * g + l
                copies.append(
                    pltpu.async_copy(
                        tiles_hbm.at[tvec[l], svec[l]],
                        outb.at[j, pl.ds(0, 64)],
                        sem,
                    ))
        for c in copies:
            c.wait()
        pltpu.sync_copy(outb, out_hbm.at[pl.ds(base, b_per_w)])

    return gather_kernel


def kernel(indices, table):
    (batch,) = indices.shape
    num_emb, dim = table.shape
    info = plsc.get_sparse_core_info()
    num_workers = info.num_cores * info.num_subcores
    b_per_w = batch // num_workers
    tiles = table.reshape(num_emb // 8, 8, dim)
    idxr = indices.astype(jnp.int32).reshape(num_workers, b_per_w // 128, 128)
    idx_pad = jnp.concatenate([idxr, jnp.zeros_like(idxr)], axis=1)
    fn = _make_sc_gather(info.num_cores, num_workers, b_per_w)
    return fn(tiles, idx_pad)[:, :dim]
